# baseline (speedup 1.0000x reference)
import jax
import jax.numpy as jnp
from jax.experimental import pallas as pl

N = 50000


def _sage(x_src, x_dst, ei, Wl, Wr, b, n_dst):
  row, col = ei[0], ei[1]
  msg = jnp.take(x_src, row, axis=0)
  s = jax.ops.segment_sum(msg, col, num_segments=n_dst)
  cnt = jax.ops.segment_sum(jnp.ones((ei.shape[1],), x_src.dtype), col,
                            num_segments=n_dst)
  mean = s / jnp.maximum(cnt, 1.0)[:, None]
  return mean @ Wl + x_dst @ Wr + b


def _ident_kernel(x_ref, o_ref):
  o_ref[...] = x_ref[...]


def kernel(x_user, x_item, edge_index_ui, edge_index_iu, edge_label_index,
           c1_ui_Wl, c1_ui_Wr, c1_ui_b, c1_iu_Wl, c1_iu_Wr, c1_iu_b,
           c2_ui_Wl, c2_ui_Wr, c2_ui_b, c2_iu_Wl, c2_iu_Wr, c2_iu_b,
           lin1_W, lin1_b, lin2_W, lin2_b):
  h_user = jax.nn.relu(_sage(x_item, x_user, edge_index_iu, c1_iu_Wl,
                             c1_iu_Wr, c1_iu_b, N))
  h_item = jax.nn.relu(_sage(x_user, x_item, edge_index_ui, c1_ui_Wl,
                             c1_ui_Wr, c1_ui_b, N))
  z_user = _sage(h_item, h_user, edge_index_iu, c2_iu_Wl, c2_iu_Wr, c2_iu_b, N)
  z_item = _sage(h_user, h_item, edge_index_ui, c2_ui_Wl, c2_ui_Wr, c2_ui_b, N)
  row, col = edge_label_index[0], edge_label_index[1]
  z = jnp.concatenate([jnp.take(z_user, row, axis=0),
                       jnp.take(z_item, col, axis=0)], axis=-1)
  z = jax.nn.relu(z @ lin1_W + lin1_b)
  z = z @ lin2_W + lin2_b
  z = z.reshape(-1, 1000)
  z = pl.pallas_call(
      _ident_kernel,
      out_shape=jax.ShapeDtypeStruct(z.shape, z.dtype),
  )(z)
  return z.reshape(-1)


# speedup vs baseline: 2.2622x; 2.2622x over previous
import dataclasses
import functools

import jax
import jax.numpy as jnp
from jax import lax
from jax.experimental import pallas as pl
from jax.experimental.pallas import tpu as pltpu
from jax.experimental.pallas import tpu_sc as plsc

N = 50000
D = 128
E = 300000
B = 100000

NTILES = 16
GRP = 128
EPT = 18816
E_PAD = NTILES * EPT
E_EXT = E_PAD + GRP
SEL_CAP = EPT + GRP
NGRP = EPT // GRP
CBLK = 1344
NBLK = EPT // CBLK
CHUNK = 6272
NCHUNK = 8
NP = NCHUNK * CHUNK
DUMMY_SPREAD = 256
ACC_ROWS = CHUNK + DUMMY_SPREAD
ZROWS = ACC_ROWS // NTILES
RPT = CHUNK // NTILES

BH = 106496
DEC_TOT = 2 * BH
DEC_GPW = 52
DEC_NBUF = 4

_mesh = plsc.VectorSubcoreMesh(core_axis_name="c", subcore_axis_name="s")

_sc_params = pltpu.CompilerParams()
if "needs_layout_passes" in pltpu.CompilerParams.__dataclass_fields__:
  _sc_params = dataclasses.replace(_sc_params, needs_layout_passes=False)


def _sage_sc_body(src_hbm, srcid_hbm, dstid_hbm, zf_hbm,
                  sums_hbm,
                  selv, colb0, colb1,
                  cids0, cids1, sids0, sids1, offs0, offs1, sidx0, sidx1,
                  rows0, rows1,
                  acc_sh,
                  colsem, isem, gsem, fsem, zsem):
  c = lax.axis_index("c")
  s = lax.axis_index("s")
  soff = (1 - c) * N
  ebase = c * E_EXT + s * EPT
  iota = lax.iota(jnp.int32, 16)
  colb = (colb0, colb1)
  cids = (cids0, cids1)
  sids = (sids0, sids1)
  offs = (offs0, offs1)
  sidx = (sidx0, sidx1)
  rows = (rows0, rows1)

  def col_dma(blk, b):
    src = dstid_hbm.at[pl.ds(ebase + blk * CBLK, CBLK)]
    return pltpu.make_async_copy(src, colb[b], colsem.at[b])

  def idx_dmas(g, b):
    sel = selv.at[pl.ds(g * GRP, GRP)]
    return (pltpu.make_async_copy(dstid_hbm.at[sel], cids[b], isem.at[b]),
            pltpu.make_async_copy(srcid_hbm.at[sel], sids[b], isem.at[b]))

  def feat_dma(b):
    return pltpu.make_async_copy(src_hbm.at[sidx[b]], rows[b], gsem.at[b])

  def scat_dmas(b):
    return (pltpu.make_async_copy(rows[b], acc_sh.at[offs[b]], fsem.at[b]),)

  @pl.loop(0, NCHUNK)
  def _(chunk):
    base = chunk * CHUNK
    plsc.subcore_barrier()
    zf_cp = pltpu.make_async_copy(zf_hbm, rows[0], zsem)
    zf_cp.start()

    col_dma(jnp.int32(0), 0).start()

    def scan_vec(blk, b):
      def body(i, p):
        col = colb[b][pl.ds(i * 16, 16)]
        u = col - base
        m = (u >= 0) & (u < CHUNK)
        cs = plsc.cumsum(m.astype(jnp.int32))
        ev = (ebase + blk * CBLK) + i * 16 + iota
        plsc.store_scatter(selv, [p + cs - 1], ev, mask=m)
        return p + jnp.sum(m.astype(jnp.int32))
      return body

    def scan_pair(bp, p):
      for b in range(2):
        blk = bp * 2 + b
        col_dma(blk, b).wait()
        nxt = blk + 1

        @pl.when(nxt < NBLK)
        def _():
          col_dma(nxt, 1 - b).start()

        p = lax.fori_loop(0, CBLK // 16, scan_vec(blk, b), p)
      return p

    p = lax.fori_loop(0, NBLK // 2, scan_pair, jnp.int32(0))

    for j in range(GRP // 16):
      selv[pl.ds(p + 16 * j, 16)] = (c * E_EXT + E_PAD + 16 * j) + iota
    ng = (p + (GRP - 1)) // GRP

    zf_cp.wait()
    z0 = s * ZROWS
    for (r0, nr) in ((0, 128), (128, 128), (256, 128), (384, ZROWS - 384)):
      pltpu.async_copy(rows[0].at[pl.ds(0, nr)],
                       acc_sh.at[pl.ds(z0 + r0, nr)], zsem)
    for (r0, nr) in ((0, 128), (128, 128), (256, 128), (384, ZROWS - 384)):
      pltpu.make_async_copy(rows[0].at[pl.ds(0, nr)],
                            acc_sh.at[pl.ds(z0 + r0, nr)], zsem).wait()
    plsc.subcore_barrier()

    for g0 in range(2):
      @pl.when(g0 < ng)
      def _():
        for d in idx_dmas(jnp.int32(g0), g0):
          d.start()

    def step(st, carry):
      for b in range(2):
        g = st * 2 + b

        @pl.when((g >= 2) & (g - 2 < ng))
        def _():
          for d in scat_dmas(b):
            d.wait()

        @pl.when(g < ng)
        def _():
          for d in idx_dmas(g, b):
            d.wait()
          for j in range(GRP // 16):
            cid = cids[b][pl.ds(16 * j, 16)]
            u = cid - base
            ok = (u >= 0) & (u < CHUNK)
            offs[b][pl.ds(16 * j, 16)] = jnp.where(
                ok, u, CHUNK + (cid & (DUMMY_SPREAD - 1)))
            sidx[b][pl.ds(16 * j, 16)] = sids[b][pl.ds(16 * j, 16)] + soff
          feat_dma(b).start()

        @pl.when(g + 2 < ng)
        def _():
          for d in idx_dmas(g + 2, b):
            d.start()

        @pl.when(g < ng)
        def _():
          feat_dma(b).wait()
          for d in scat_dmas(b):
            d.start(add=True)
      return carry

    nsteps = (ng + 3) // 2
    lax.fori_loop(0, nsteps, step, jnp.int32(0))

    plsc.subcore_barrier()
    pltpu.sync_copy(acc_sh.at[pl.ds(s * RPT, RPT)],
                    sums_hbm.at[c, pl.ds(base + s * RPT, RPT)])


def _sage_sc(src_flat, srcid_flat, dstid_flat):
  zf = jnp.zeros((GRP, D), jnp.float32)
  kern = pl.kernel(
      _sage_sc_body,
      out_type=jax.ShapeDtypeStruct((2, NP, D), jnp.float32),
      mesh=_mesh,
      compiler_params=_sc_params,
      scratch_types=[
          pltpu.VMEM((SEL_CAP,), jnp.int32),
          pltpu.VMEM((CBLK,), jnp.int32),
          pltpu.VMEM((CBLK,), jnp.int32),
          pltpu.VMEM((GRP,), jnp.int32),
          pltpu.VMEM((GRP,), jnp.int32),
          pltpu.VMEM((GRP,), jnp.int32),
          pltpu.VMEM((GRP,), jnp.int32),
          pltpu.VMEM((GRP,), jnp.int32),
          pltpu.VMEM((GRP,), jnp.int32),
          pltpu.VMEM((GRP,), jnp.int32),
          pltpu.VMEM((GRP,), jnp.int32),
          pltpu.VMEM((GRP, D), jnp.float32),
          pltpu.VMEM((GRP, D), jnp.float32),
          pltpu.VMEM_SHARED((ACC_ROWS, D), jnp.float32),
          pltpu.SemaphoreType.DMA((2,)),
          pltpu.SemaphoreType.DMA((2,)),
          pltpu.SemaphoreType.DMA((2,)),
          pltpu.SemaphoreType.DMA((2,)),
          pltpu.SemaphoreType.DMA,
      ],
  )
  return kern(src_flat, srcid_flat, dstid_flat, zf)


def _dec_sc_body(z_hbm, idx_hbm, out_hbm, idx_v, rows0, rows1, rows2, rows3,
                 gsem, osem):
  c = lax.axis_index("c")
  s = lax.axis_index("s")
  w = s * 2 + c
  rows = (rows0, rows1, rows2, rows3)
  pltpu.sync_copy(idx_hbm.at[w], idx_v)

  def gather_dma(g, b):
    return pltpu.make_async_copy(z_hbm.at[idx_v.at[g]], rows[b], gsem.at[b])

  def out_dma(g, b):
    dst = out_hbm.at[pl.ds(w * (DEC_GPW * GRP) + g * GRP, GRP)]
    return pltpu.make_async_copy(rows[b], dst, osem.at[b])

  for b in range(DEC_NBUF):
    gather_dma(jnp.int32(b), b).start()

  @pl.loop(0, DEC_GPW // DEC_NBUF)
  def _(it):
    for b in range(DEC_NBUF):
      g = it * DEC_NBUF + b
      gather_dma(g, b).wait()
      out_dma(g, b).start()
      out_dma(g, b).wait()
      g_next = g + DEC_NBUF

      @pl.when(g_next < DEC_GPW)
      def _():
        gather_dma(g_next, b).start()


def _dec_sc(z_flat, dec_idx):
  kern = pl.kernel(
      _dec_sc_body,
      out_type=jax.ShapeDtypeStruct((DEC_TOT, D), jnp.float32),
      mesh=_mesh,
      scratch_types=[
          pltpu.VMEM((DEC_GPW, GRP), jnp.int32),
          pltpu.VMEM((GRP, D), jnp.float32),
          pltpu.VMEM((GRP, D), jnp.float32),
          pltpu.VMEM((GRP, D), jnp.float32),
          pltpu.VMEM((GRP, D), jnp.float32),
          pltpu.SemaphoreType.DMA((DEC_NBUF,)),
          pltpu.SemaphoreType.DMA((DEC_NBUF,)),
      ],
  )
  return kern(z_flat, dec_idx)


_TCR = 1000


def _sage_tc_kernel(relu, sum_ref, cnt_ref, x_ref, wl_ref, wr_ref, b_ref,
                    o_ref):
  cnt = cnt_ref[0, :, 0:1]
  inv = 1.0 / jnp.maximum(cnt, 1.0)
  mean = sum_ref[0] * inv
  acc = jnp.dot(mean, wl_ref[0], preferred_element_type=jnp.float32)
  acc = acc + jnp.dot(x_ref[0], wr_ref[0], preferred_element_type=jnp.float32)
  acc = acc + b_ref[0, 0]
  if relu:
    acc = jnp.maximum(acc, 0.0)
  o_ref[0] = acc


def _sage_tc(sums, cnts, x_stack, wl, wr, b, relu):
  grid = (2, N // _TCR)
  return pl.pallas_call(
      functools.partial(_sage_tc_kernel, relu),
      grid=grid,
      in_specs=[
          pl.BlockSpec((1, _TCR, D), lambda t, i: (t, i, 0)),
          pl.BlockSpec((1, _TCR, D), lambda t, i: (t, i, 0)),
          pl.BlockSpec((1, _TCR, D), lambda t, i: (t, i, 0)),
          pl.BlockSpec((1, D, D), lambda t, i: (t, 0, 0)),
          pl.BlockSpec((1, D, D), lambda t, i: (t, 0, 0)),
          pl.BlockSpec((1, 1, D), lambda t, i: (t, 0, 0)),
      ],
      out_specs=pl.BlockSpec((1, _TCR, D), lambda t, i: (t, i, 0)),
      out_shape=jax.ShapeDtypeStruct((2, N, D), jnp.float32),
  )(sums, cnts, x_stack, wl, wr, b)


_DECR = 1024


def _dec_tc_kernel(gu_ref, gi_ref, w1a_ref, w1b_ref, b1_ref, w2_ref, b2_ref,
                   o_ref):
  t = jnp.dot(gu_ref[...], w1a_ref[...], preferred_element_type=jnp.float32)
  t = t + jnp.dot(gi_ref[...], w1b_ref[...], preferred_element_type=jnp.float32)
  t = jnp.maximum(t + b1_ref[0], 0.0)
  y = jnp.sum(t * w2_ref[0], axis=1, keepdims=True) + b2_ref[0, 0]
  o_ref[...] = y


def _dec_tc(gathered, lin1_W, lin1_b, lin2_W, lin2_b):
  w1a = lin1_W[:D]
  w1b = lin1_W[D:]
  b1 = lin1_b.reshape(1, D)
  w2 = lin2_W.reshape(1, D)
  b2 = lin2_b.reshape(1, 1)
  grid = (pl.cdiv(B, _DECR),)
  off = BH // _DECR
  return pl.pallas_call(
      _dec_tc_kernel,
      grid=grid,
      in_specs=[
          pl.BlockSpec((_DECR, D), lambda i: (i, 0)),
          pl.BlockSpec((_DECR, D), lambda i: (i + off, 0)),
          pl.BlockSpec((D, D), lambda i: (0, 0)),
          pl.BlockSpec((D, D), lambda i: (0, 0)),
          pl.BlockSpec((1, D), lambda i: (0, 0)),
          pl.BlockSpec((1, D), lambda i: (0, 0)),
          pl.BlockSpec(memory_space=pltpu.SMEM),
      ],
      out_specs=pl.BlockSpec((_DECR, 1), lambda i: (i, 0)),
      out_shape=jax.ShapeDtypeStruct((B, 1), jnp.float32),
  )(gathered, gathered, w1a, w1b, b1, w2, b2)


def _pad_to(a, n, val):
  return jnp.concatenate(
      [a, jnp.full((n - a.shape[0],), val, a.dtype)])


def kernel(x_user, x_item, edge_index_ui, edge_index_iu, edge_label_index,
           c1_ui_Wl, c1_ui_Wr, c1_ui_b, c1_iu_Wl, c1_iu_Wr, c1_iu_b,
           c2_ui_Wl, c2_ui_Wr, c2_ui_b, c2_iu_Wl, c2_iu_Wr, c2_iu_b,
           lin1_W, lin1_b, lin2_W, lin2_b):
  i32 = jnp.int32
  iu_s = edge_index_iu[0].astype(i32)
  iu_d = edge_index_iu[1].astype(i32)
  ui_s = edge_index_ui[0].astype(i32)
  ui_d = edge_index_ui[1].astype(i32)

  srcid_flat = jnp.concatenate(
      [_pad_to(iu_s, E_EXT, 0), _pad_to(ui_s, E_EXT, 0)])
  dstid_flat = jnp.concatenate(
      [_pad_to(iu_d, E_EXT, -1), _pad_to(ui_d, E_EXT, -1)])

  x_flat = jnp.concatenate([x_user, x_item], axis=0)
  x_stack = x_flat.reshape(2, N, D)

  wl1 = jnp.stack([c1_iu_Wl, c1_ui_Wl])
  wr1 = jnp.stack([c1_iu_Wr, c1_ui_Wr])
  b1s = jnp.stack([c1_iu_b, c1_ui_b]).reshape(2, 1, D)
  wl2 = jnp.stack([c2_iu_Wl, c2_ui_Wl])
  wr2 = jnp.stack([c2_iu_Wr, c2_ui_Wr])
  b2s = jnp.stack([c2_iu_b, c2_ui_b]).reshape(2, 1, D)

  ones_flat = jnp.ones((2 * N, D), jnp.float32)
  cnts = _sage_sc(ones_flat, srcid_flat, dstid_flat)
  sums1 = _sage_sc(x_flat, srcid_flat, dstid_flat)
  h = _sage_tc(sums1, cnts, x_stack, wl1, wr1, b1s, relu=True)

  sums2 = _sage_sc(h.reshape(2 * N, D), srcid_flat, dstid_flat)
  z = _sage_tc(sums2, cnts, h, wl2, wr2, b2s, relu=False)

  idx_u = _pad_to(edge_label_index[0].astype(i32), BH, 0)
  idx_i = _pad_to(edge_label_index[1].astype(i32), BH, 0) + N
  dec_idx = jnp.concatenate([idx_u, idx_i]).reshape(32, DEC_GPW, GRP)

  gathered = _dec_sc(z.reshape(2 * N, D), dec_idx)
  out = _dec_tc(gathered, lin1_W, lin1_b, lin2_W, lin2_b)
  return out.reshape(-1)


# speedup vs baseline: 2.5908x; 1.1453x over previous
import dataclasses
import functools

import jax
import jax.numpy as jnp
from jax import lax
from jax.experimental import pallas as pl
from jax.experimental.pallas import tpu as pltpu
from jax.experimental.pallas import tpu_sc as plsc

N = 50000
D = 128
E = 300000
B = 100000

NTILES = 16
GRP = 128
EPT = 18816
E_PAD = NTILES * EPT
E_EXT = E_PAD + GRP
SEL_CAP = EPT + GRP
NGRP = EPT // GRP
CBLK = 1344
NBLK = EPT // CBLK
CHUNK = 6272
NCHUNK = 8
NP = NCHUNK * CHUNK
DUMMY_SPREAD = 256
ACC_ROWS = CHUNK + DUMMY_SPREAD
ZROWS = ACC_ROWS // NTILES
RPT = CHUNK // NTILES

BH = 106496
DEC_TOT = 2 * BH
DEC_GPW = 52
DEC_NBUF = 4

_mesh = plsc.VectorSubcoreMesh(core_axis_name="c", subcore_axis_name="s")

_sc_params = pltpu.CompilerParams()
if "needs_layout_passes" in pltpu.CompilerParams.__dataclass_fields__:
  _sc_params = dataclasses.replace(_sc_params, needs_layout_passes=False)


def _sage_sc_body(count_only, src_hbm, srcid_hbm, dstid_hbm, zf_hbm,
                  sums_hbm,
                  selv, colb0, colb1,
                  cids0, cids1, sids0, sids1, offs0, offs1, sidx0, sidx1,
                  rows0, rows1,
                  acc_sh,
                  colsem, isem, gsem, fsem, zsem):
  c = lax.axis_index("c")
  s = lax.axis_index("s")
  soff = (1 - c) * N
  ebase = c * E_EXT + s * EPT
  iota = lax.iota(jnp.int32, 16)
  colb = (colb0, colb1)
  cids = (cids0, cids1)
  sids = (sids0, sids1)
  offs = (offs0, offs1)
  sidx = (sidx0, sidx1)
  rows = (rows0, rows1)

  if count_only:
    pltpu.sync_copy(src_hbm.at[pl.ds(0, GRP)], rows[1])

  def col_dma(blk, b):
    src = dstid_hbm.at[pl.ds(ebase + blk * CBLK, CBLK)]
    return pltpu.make_async_copy(src, colb[b], colsem.at[b])

  def idx_dmas(g, b):
    sel = selv.at[pl.ds(g * GRP, GRP)]
    dmas = [pltpu.make_async_copy(dstid_hbm.at[sel], cids[b], isem.at[b])]
    if not count_only:
      dmas.append(
          pltpu.make_async_copy(srcid_hbm.at[sel], sids[b], isem.at[b]))
    return dmas

  def feat_dma(b):
    return pltpu.make_async_copy(src_hbm.at[sidx[b]], rows[b], gsem.at[b])

  def scat_dmas(b):
    srcbuf = rows[1] if count_only else rows[b]
    return (pltpu.make_async_copy(srcbuf, acc_sh.at[offs[b]], fsem.at[b]),)

  @pl.loop(0, NCHUNK)
  def _(chunk):
    base = chunk * CHUNK
    plsc.subcore_barrier()
    zf_cp = pltpu.make_async_copy(zf_hbm, rows[0], zsem)
    zf_cp.start()

    col_dma(jnp.int32(0), 0).start()

    def scan_vec(blk, b):
      def body(i, p):
        col = colb[b][pl.ds(i * 16, 16)]
        u = col - base
        m = (u >= 0) & (u < CHUNK)
        cs = plsc.cumsum(m.astype(jnp.int32))
        ev = (ebase + blk * CBLK) + i * 16 + iota
        plsc.store_scatter(selv, [p + cs - 1], ev, mask=m)
        return p + jnp.sum(m.astype(jnp.int32))
      return body

    def scan_pair(bp, p):
      for b in range(2):
        blk = bp * 2 + b
        col_dma(blk, b).wait()
        nxt = blk + 1

        @pl.when(nxt < NBLK)
        def _():
          col_dma(nxt, 1 - b).start()

        p = lax.fori_loop(0, CBLK // 16, scan_vec(blk, b), p)
      return p

    p = lax.fori_loop(0, NBLK // 2, scan_pair, jnp.int32(0))

    for j in range(GRP // 16):
      selv[pl.ds(p + 16 * j, 16)] = (c * E_EXT + E_PAD + 16 * j) + iota
    ng = (p + (GRP - 1)) // GRP

    zf_cp.wait()
    z0 = s * ZROWS
    for (r0, nr) in ((0, 128), (128, 128), (256, 128), (384, ZROWS - 384)):
      pltpu.async_copy(rows[0].at[pl.ds(0, nr)],
                       acc_sh.at[pl.ds(z0 + r0, nr)], zsem)
    for (r0, nr) in ((0, 128), (128, 128), (256, 128), (384, ZROWS - 384)):
      pltpu.make_async_copy(rows[0].at[pl.ds(0, nr)],
                            acc_sh.at[pl.ds(z0 + r0, nr)], zsem).wait()
    plsc.subcore_barrier()

    for g0 in range(2):
      @pl.when(g0 < ng)
      def _():
        for d in idx_dmas(jnp.int32(g0), g0):
          d.start()

    def step(st, carry):
      for b in range(2):
        g = st * 2 + b

        @pl.when((g >= 2) & (g - 2 < ng))
        def _():
          for d in scat_dmas(b):
            d.wait()

        @pl.when(g < ng)
        def _():
          for d in idx_dmas(g, b):
            d.wait()
          for j in range(GRP // 16):
            cid = cids[b][pl.ds(16 * j, 16)]
            u = cid - base
            ok = (u >= 0) & (u < CHUNK)
            offs[b][pl.ds(16 * j, 16)] = jnp.where(
                ok, u, CHUNK + (cid & (DUMMY_SPREAD - 1)))
            if not count_only:
              sidx[b][pl.ds(16 * j, 16)] = sids[b][pl.ds(16 * j, 16)] + soff
          if not count_only:
            feat_dma(b).start()

        @pl.when(g + 2 < ng)
        def _():
          for d in idx_dmas(g + 2, b):
            d.start()

        @pl.when(g < ng)
        def _():
          if not count_only:
            feat_dma(b).wait()
          for d in scat_dmas(b):
            d.start(add=True)
      return carry

    nsteps = (ng + 3) // 2
    lax.fori_loop(0, nsteps, step, jnp.int32(0))

    plsc.subcore_barrier()
    pltpu.sync_copy(acc_sh.at[pl.ds(s * RPT, RPT)],
                    sums_hbm.at[c, pl.ds(base + s * RPT, RPT)])


def _sage_sc(src_flat, srcid_flat, dstid_flat, count_only=False):
  zf = jnp.zeros((GRP, D), jnp.float32)
  kern = pl.kernel(
      functools.partial(_sage_sc_body, count_only),
      out_type=jax.ShapeDtypeStruct((2, NP, D), jnp.float32),
      mesh=_mesh,
      compiler_params=_sc_params,
      scratch_types=[
          pltpu.VMEM((SEL_CAP,), jnp.int32),
          pltpu.VMEM((CBLK,), jnp.int32),
          pltpu.VMEM((CBLK,), jnp.int32),
          pltpu.VMEM((GRP,), jnp.int32),
          pltpu.VMEM((GRP,), jnp.int32),
          pltpu.VMEM((GRP,), jnp.int32),
          pltpu.VMEM((GRP,), jnp.int32),
          pltpu.VMEM((GRP,), jnp.int32),
          pltpu.VMEM((GRP,), jnp.int32),
          pltpu.VMEM((GRP,), jnp.int32),
          pltpu.VMEM((GRP,), jnp.int32),
          pltpu.VMEM((GRP, D), jnp.float32),
          pltpu.VMEM((GRP, D), jnp.float32),
          pltpu.VMEM_SHARED((ACC_ROWS, D), jnp.float32),
          pltpu.SemaphoreType.DMA((2,)),
          pltpu.SemaphoreType.DMA((2,)),
          pltpu.SemaphoreType.DMA((2,)),
          pltpu.SemaphoreType.DMA((2,)),
          pltpu.SemaphoreType.DMA,
      ],
  )
  return kern(src_flat, srcid_flat, dstid_flat, zf)


def _dec_sc_body(z_hbm, idx_hbm, out_hbm, idx_v, rows0, rows1, rows2, rows3,
                 gsem, osem):
  c = lax.axis_index("c")
  s = lax.axis_index("s")
  w = s * 2 + c
  rows = (rows0, rows1, rows2, rows3)
  pltpu.sync_copy(idx_hbm.at[w], idx_v)

  def gather_dma(g, b):
    return pltpu.make_async_copy(z_hbm.at[idx_v.at[g]], rows[b], gsem.at[b])

  def out_dma(g, b):
    dst = out_hbm.at[pl.ds(w * (DEC_GPW * GRP) + g * GRP, GRP)]
    return pltpu.make_async_copy(rows[b], dst, osem.at[b])

  for b in range(DEC_NBUF):
    gather_dma(jnp.int32(b), b).start()

  @pl.loop(0, DEC_GPW // DEC_NBUF)
  def _(it):
    for b in range(DEC_NBUF):
      g = it * DEC_NBUF + b
      gather_dma(g, b).wait()
      out_dma(g, b).start()
      out_dma(g, b).wait()
      g_next = g + DEC_NBUF

      @pl.when(g_next < DEC_GPW)
      def _():
        gather_dma(g_next, b).start()


def _dec_sc(z_flat, dec_idx):
  kern = pl.kernel(
      _dec_sc_body,
      out_type=jax.ShapeDtypeStruct((DEC_TOT, D), jnp.float32),
      mesh=_mesh,
      scratch_types=[
          pltpu.VMEM((DEC_GPW, GRP), jnp.int32),
          pltpu.VMEM((GRP, D), jnp.float32),
          pltpu.VMEM((GRP, D), jnp.float32),
          pltpu.VMEM((GRP, D), jnp.float32),
          pltpu.VMEM((GRP, D), jnp.float32),
          pltpu.SemaphoreType.DMA((DEC_NBUF,)),
          pltpu.SemaphoreType.DMA((DEC_NBUF,)),
      ],
  )
  return kern(z_flat, dec_idx)


_TCR = 1000


def _sage_tc_kernel(relu, sum_ref, cnt_ref, x_ref, wl_ref, wr_ref, b_ref,
                    o_ref):
  cnt = cnt_ref[0, :, 0:1]
  inv = 1.0 / jnp.maximum(cnt, 1.0)
  mean = sum_ref[0] * inv
  acc = jnp.dot(mean, wl_ref[0], preferred_element_type=jnp.float32)
  acc = acc + jnp.dot(x_ref[0], wr_ref[0], preferred_element_type=jnp.float32)
  acc = acc + b_ref[0, 0]
  if relu:
    acc = jnp.maximum(acc, 0.0)
  o_ref[0] = acc


def _sage_tc(sums, cnts, x_stack, wl, wr, b, relu):
  grid = (2, N // _TCR)
  return pl.pallas_call(
      functools.partial(_sage_tc_kernel, relu),
      grid=grid,
      in_specs=[
          pl.BlockSpec((1, _TCR, D), lambda t, i: (t, i, 0)),
          pl.BlockSpec((1, _TCR, D), lambda t, i: (t, i, 0)),
          pl.BlockSpec((1, _TCR, D), lambda t, i: (t, i, 0)),
          pl.BlockSpec((1, D, D), lambda t, i: (t, 0, 0)),
          pl.BlockSpec((1, D, D), lambda t, i: (t, 0, 0)),
          pl.BlockSpec((1, 1, D), lambda t, i: (t, 0, 0)),
      ],
      out_specs=pl.BlockSpec((1, _TCR, D), lambda t, i: (t, i, 0)),
      out_shape=jax.ShapeDtypeStruct((2, N, D), jnp.float32),
  )(sums, cnts, x_stack, wl, wr, b)


_DECR = 1024


def _dec_tc_kernel(gu_ref, gi_ref, w1a_ref, w1b_ref, b1_ref, w2_ref, b2_ref,
                   o_ref):
  t = jnp.dot(gu_ref[...], w1a_ref[...], preferred_element_type=jnp.float32)
  t = t + jnp.dot(gi_ref[...], w1b_ref[...], preferred_element_type=jnp.float32)
  t = jnp.maximum(t + b1_ref[0], 0.0)
  y = jnp.sum(t * w2_ref[0], axis=1, keepdims=True) + b2_ref[0, 0]
  o_ref[...] = y


def _dec_tc(gathered, lin1_W, lin1_b, lin2_W, lin2_b):
  w1a = lin1_W[:D]
  w1b = lin1_W[D:]
  b1 = lin1_b.reshape(1, D)
  w2 = lin2_W.reshape(1, D)
  b2 = lin2_b.reshape(1, 1)
  grid = (pl.cdiv(B, _DECR),)
  off = BH // _DECR
  return pl.pallas_call(
      _dec_tc_kernel,
      grid=grid,
      in_specs=[
          pl.BlockSpec((_DECR, D), lambda i: (i, 0)),
          pl.BlockSpec((_DECR, D), lambda i: (i + off, 0)),
          pl.BlockSpec((D, D), lambda i: (0, 0)),
          pl.BlockSpec((D, D), lambda i: (0, 0)),
          pl.BlockSpec((1, D), lambda i: (0, 0)),
          pl.BlockSpec((1, D), lambda i: (0, 0)),
          pl.BlockSpec(memory_space=pltpu.SMEM),
      ],
      out_specs=pl.BlockSpec((_DECR, 1), lambda i: (i, 0)),
      out_shape=jax.ShapeDtypeStruct((B, 1), jnp.float32),
  )(gathered, gathered, w1a, w1b, b1, w2, b2)


def _pad_to(a, n, val):
  return jnp.concatenate(
      [a, jnp.full((n - a.shape[0],), val, a.dtype)])


def kernel(x_user, x_item, edge_index_ui, edge_index_iu, edge_label_index,
           c1_ui_Wl, c1_ui_Wr, c1_ui_b, c1_iu_Wl, c1_iu_Wr, c1_iu_b,
           c2_ui_Wl, c2_ui_Wr, c2_ui_b, c2_iu_Wl, c2_iu_Wr, c2_iu_b,
           lin1_W, lin1_b, lin2_W, lin2_b):
  i32 = jnp.int32
  iu_s = edge_index_iu[0].astype(i32)
  iu_d = edge_index_iu[1].astype(i32)
  ui_s = edge_index_ui[0].astype(i32)
  ui_d = edge_index_ui[1].astype(i32)

  srcid_flat = jnp.concatenate(
      [_pad_to(iu_s, E_EXT, 0), _pad_to(ui_s, E_EXT, 0)])
  dstid_flat = jnp.concatenate(
      [_pad_to(iu_d, E_EXT, -1), _pad_to(ui_d, E_EXT, -1)])

  x_flat = jnp.concatenate([x_user, x_item], axis=0)
  x_stack = x_flat.reshape(2, N, D)

  wl1 = jnp.stack([c1_iu_Wl, c1_ui_Wl])
  wr1 = jnp.stack([c1_iu_Wr, c1_ui_Wr])
  b1s = jnp.stack([c1_iu_b, c1_ui_b]).reshape(2, 1, D)
  wl2 = jnp.stack([c2_iu_Wl, c2_ui_Wl])
  wr2 = jnp.stack([c2_iu_Wr, c2_ui_Wr])
  b2s = jnp.stack([c2_iu_b, c2_ui_b]).reshape(2, 1, D)

  ones_flat = jnp.ones((GRP, D), jnp.float32)
  cnts = _sage_sc(ones_flat, srcid_flat, dstid_flat, count_only=True)
  sums1 = _sage_sc(x_flat, srcid_flat, dstid_flat)
  h = _sage_tc(sums1, cnts, x_stack, wl1, wr1, b1s, relu=True)

  sums2 = _sage_sc(h.reshape(2 * N, D), srcid_flat, dstid_flat)
  z = _sage_tc(sums2, cnts, h, wl2, wr2, b2s, relu=False)

  idx_u = _pad_to(edge_label_index[0].astype(i32), BH, 0)
  idx_i = _pad_to(edge_label_index[1].astype(i32), BH, 0) + N
  dec_idx = jnp.concatenate([idx_u, idx_i]).reshape(32, DEC_GPW, GRP)

  gathered = _dec_sc(z.reshape(2 * N, D), dec_idx)
  out = _dec_tc(gathered, lin1_W, lin1_b, lin2_W, lin2_b)
  return out.reshape(-1)


# speedup vs baseline: 2.6945x; 1.0400x over previous
import dataclasses
import functools

import jax
import jax.numpy as jnp
from jax import lax
from jax.experimental import pallas as pl
from jax.experimental.pallas import tpu as pltpu
from jax.experimental.pallas import tpu_sc as plsc

N = 50000
D = 128
E = 300000
B = 100000

NTILES = 16
GRP = 128
EPT = 18816
E_PAD = NTILES * EPT
E_EXT = E_PAD + GRP
SEL_CAP = EPT + GRP
NGRP = EPT // GRP
CBLK = 1344
NBLK = EPT // CBLK
CHUNK = 6272
NCHUNK = 8
NP = NCHUNK * CHUNK
DUMMY_SPREAD = 256
ACC_ROWS = CHUNK + DUMMY_SPREAD
ZROWS = ACC_ROWS // NTILES
RPT = CHUNK // NTILES

BH = 106496
DEC_TOT = 2 * BH
DEC_GPW = 52
DEC_NBUF = 4

_mesh = plsc.VectorSubcoreMesh(core_axis_name="c", subcore_axis_name="s")

_sc_params = pltpu.CompilerParams()
if "needs_layout_passes" in pltpu.CompilerParams.__dataclass_fields__:
  _sc_params = dataclasses.replace(_sc_params, needs_layout_passes=False)


def _sage_sc_body(count_only, src_hbm, srcid_hbm, dstid_hbm, zf_hbm,
                  sums_hbm,
                  selv, colb0, colb1,
                  cids0, cids1, sids0, sids1, offs0, offs1, sidx0, sidx1,
                  rows0, rows1,
                  acc_sh,
                  colsem, isem, gsem, fsem, zsem):
  c = lax.axis_index("c")
  s = lax.axis_index("s")
  soff = (1 - c) * N
  ebase = c * E_EXT + s * EPT
  iota = lax.iota(jnp.int32, 16)
  colb = (colb0, colb1)
  cids = (cids0, cids1)
  sids = (sids0, sids1)
  offs = (offs0, offs1)
  sidx = (sidx0, sidx1)
  rows = (rows0, rows1)

  if count_only:
    pltpu.sync_copy(src_hbm.at[pl.ds(0, GRP)], rows[1])

  def col_dma(blk, b):
    src = dstid_hbm.at[pl.ds(ebase + blk * CBLK, CBLK)]
    return pltpu.make_async_copy(src, colb[b], colsem.at[b])

  def idx_dmas(g, b):
    sel = selv.at[pl.ds(g * GRP, GRP)]
    dmas = [pltpu.make_async_copy(dstid_hbm.at[sel], cids[b], isem.at[b])]
    if not count_only:
      dmas.append(
          pltpu.make_async_copy(srcid_hbm.at[sel], sids[b], isem.at[b]))
    return dmas

  def feat_dma(b):
    return pltpu.make_async_copy(src_hbm.at[sidx[b]], rows[b], gsem.at[b])

  def scat_dmas(b):
    srcbuf = rows[1] if count_only else rows[b]
    return (pltpu.make_async_copy(srcbuf, acc_sh.at[offs[b]], fsem.at[b]),)

  @pl.loop(0, NCHUNK)
  def _(chunk):
    base = chunk * CHUNK
    plsc.subcore_barrier()
    zf_cp = pltpu.make_async_copy(zf_hbm, rows[0], zsem)
    zf_cp.start()

    col_dma(jnp.int32(0), 0).start()

    def scan_vec(blk, b):
      def body(i, p):
        col = colb[b][pl.ds(i * 16, 16)]
        u = col - base
        m = (u >= 0) & (u < CHUNK)
        cs = plsc.cumsum(m.astype(jnp.int32))
        ev = (ebase + blk * CBLK) + i * 16 + iota
        plsc.store_scatter(selv, [p + cs - 1], ev, mask=m)
        return p + jnp.sum(m.astype(jnp.int32))
      return body

    def scan_pair(bp, p):
      for b in range(2):
        blk = bp * 2 + b
        col_dma(blk, b).wait()
        nxt = blk + 1

        @pl.when(nxt < NBLK)
        def _():
          col_dma(nxt, 1 - b).start()

        p = lax.fori_loop(0, CBLK // 16, scan_vec(blk, b), p)
      return p

    p = lax.fori_loop(0, NBLK // 2, scan_pair, jnp.int32(0))

    for j in range(GRP // 16):
      selv[pl.ds(p + 16 * j, 16)] = (c * E_EXT + E_PAD + 16 * j) + iota
    ng = (p + (GRP - 1)) // GRP

    zf_cp.wait()
    z0 = s * ZROWS
    for (r0, nr) in ((0, 128), (128, 128), (256, 128), (384, ZROWS - 384)):
      pltpu.async_copy(rows[0].at[pl.ds(0, nr)],
                       acc_sh.at[pl.ds(z0 + r0, nr)], zsem)
    for (r0, nr) in ((0, 128), (128, 128), (256, 128), (384, ZROWS - 384)):
      pltpu.make_async_copy(rows[0].at[pl.ds(0, nr)],
                            acc_sh.at[pl.ds(z0 + r0, nr)], zsem).wait()
    plsc.subcore_barrier()

    for g0 in range(2):
      @pl.when(g0 < ng)
      def _():
        for d in idx_dmas(jnp.int32(g0), g0):
          d.start()

    def step(st, carry):
      for b in range(2):
        g = st * 2 + b

        @pl.when((g >= 2) & (g - 2 < ng))
        def _():
          for d in scat_dmas(b):
            d.wait()

        @pl.when(g < ng)
        def _():
          for d in idx_dmas(g, b):
            d.wait()
          for j in range(GRP // 16):
            cid = cids[b][pl.ds(16 * j, 16)]
            u = cid - base
            ok = (u >= 0) & (u < CHUNK)
            offs[b][pl.ds(16 * j, 16)] = jnp.where(
                ok, u, CHUNK + (cid & (DUMMY_SPREAD - 1)))
            if not count_only:
              sidx[b][pl.ds(16 * j, 16)] = sids[b][pl.ds(16 * j, 16)] + soff
          if not count_only:
            feat_dma(b).start()

        @pl.when(g + 2 < ng)
        def _():
          for d in idx_dmas(g + 2, b):
            d.start()

      for b in range(2):
        g = st * 2 + b

        @pl.when(g < ng)
        def _():
          if not count_only:
            feat_dma(b).wait()
          for d in scat_dmas(b):
            d.start(add=True)
      return carry

    nsteps = (ng + 3) // 2
    lax.fori_loop(0, nsteps, step, jnp.int32(0))

    plsc.subcore_barrier()
    pltpu.sync_copy(acc_sh.at[pl.ds(s * RPT, RPT)],
                    sums_hbm.at[c, pl.ds(base + s * RPT, RPT)])


def _sage_sc(src_flat, srcid_flat, dstid_flat, count_only=False):
  zf = jnp.zeros((GRP, D), jnp.float32)
  kern = pl.kernel(
      functools.partial(_sage_sc_body, count_only),
      out_type=jax.ShapeDtypeStruct((2, NP, D), jnp.float32),
      mesh=_mesh,
      compiler_params=_sc_params,
      scratch_types=[
          pltpu.VMEM((SEL_CAP,), jnp.int32),
          pltpu.VMEM((CBLK,), jnp.int32),
          pltpu.VMEM((CBLK,), jnp.int32),
          pltpu.VMEM((GRP,), jnp.int32),
          pltpu.VMEM((GRP,), jnp.int32),
          pltpu.VMEM((GRP,), jnp.int32),
          pltpu.VMEM((GRP,), jnp.int32),
          pltpu.VMEM((GRP,), jnp.int32),
          pltpu.VMEM((GRP,), jnp.int32),
          pltpu.VMEM((GRP,), jnp.int32),
          pltpu.VMEM((GRP,), jnp.int32),
          pltpu.VMEM((GRP, D), jnp.float32),
          pltpu.VMEM((GRP, D), jnp.float32),
          pltpu.VMEM_SHARED((ACC_ROWS, D), jnp.float32),
          pltpu.SemaphoreType.DMA((2,)),
          pltpu.SemaphoreType.DMA((2,)),
          pltpu.SemaphoreType.DMA((2,)),
          pltpu.SemaphoreType.DMA((2,)),
          pltpu.SemaphoreType.DMA,
      ],
  )
  return kern(src_flat, srcid_flat, dstid_flat, zf)


def _dec_sc_body(z_hbm, idx_hbm, out_hbm, idx_v, rows0, rows1, rows2, rows3,
                 gsem, osem):
  c = lax.axis_index("c")
  s = lax.axis_index("s")
  w = s * 2 + c
  rows = (rows0, rows1, rows2, rows3)
  pltpu.sync_copy(idx_hbm.at[w], idx_v)

  def gather_dma(g, b):
    return pltpu.make_async_copy(z_hbm.at[idx_v.at[g]], rows[b], gsem.at[b])

  def out_dma(g, b):
    dst = out_hbm.at[pl.ds(w * (DEC_GPW * GRP) + g * GRP, GRP)]
    return pltpu.make_async_copy(rows[b], dst, osem.at[b])

  for b in range(DEC_NBUF):
    gather_dma(jnp.int32(b), b).start()

  @pl.loop(0, DEC_GPW // DEC_NBUF)
  def _(it):
    for b in range(DEC_NBUF):
      g = it * DEC_NBUF + b
      gather_dma(g, b).wait()
      out_dma(g, b).start()
      out_dma(g, b).wait()
      g_next = g + DEC_NBUF

      @pl.when(g_next < DEC_GPW)
      def _():
        gather_dma(g_next, b).start()


def _dec_sc(z_flat, dec_idx):
  kern = pl.kernel(
      _dec_sc_body,
      out_type=jax.ShapeDtypeStruct((DEC_TOT, D), jnp.float32),
      mesh=_mesh,
      scratch_types=[
          pltpu.VMEM((DEC_GPW, GRP), jnp.int32),
          pltpu.VMEM((GRP, D), jnp.float32),
          pltpu.VMEM((GRP, D), jnp.float32),
          pltpu.VMEM((GRP, D), jnp.float32),
          pltpu.VMEM((GRP, D), jnp.float32),
          pltpu.SemaphoreType.DMA((DEC_NBUF,)),
          pltpu.SemaphoreType.DMA((DEC_NBUF,)),
      ],
  )
  return kern(z_flat, dec_idx)


_TCR = 1000


def _sage_tc_kernel(relu, sum_ref, cnt_ref, x_ref, wl_ref, wr_ref, b_ref,
                    o_ref):
  cnt = cnt_ref[0, :, 0:1]
  inv = 1.0 / jnp.maximum(cnt, 1.0)
  mean = sum_ref[0] * inv
  acc = jnp.dot(mean, wl_ref[0], preferred_element_type=jnp.float32)
  acc = acc + jnp.dot(x_ref[0], wr_ref[0], preferred_element_type=jnp.float32)
  acc = acc + b_ref[0, 0]
  if relu:
    acc = jnp.maximum(acc, 0.0)
  o_ref[0] = acc


def _sage_tc(sums, cnts, x_stack, wl, wr, b, relu):
  grid = (2, N // _TCR)
  return pl.pallas_call(
      functools.partial(_sage_tc_kernel, relu),
      grid=grid,
      in_specs=[
          pl.BlockSpec((1, _TCR, D), lambda t, i: (t, i, 0)),
          pl.BlockSpec((1, _TCR, D), lambda t, i: (t, i, 0)),
          pl.BlockSpec((1, _TCR, D), lambda t, i: (t, i, 0)),
          pl.BlockSpec((1, D, D), lambda t, i: (t, 0, 0)),
          pl.BlockSpec((1, D, D), lambda t, i: (t, 0, 0)),
          pl.BlockSpec((1, 1, D), lambda t, i: (t, 0, 0)),
      ],
      out_specs=pl.BlockSpec((1, _TCR, D), lambda t, i: (t, i, 0)),
      out_shape=jax.ShapeDtypeStruct((2, N, D), jnp.float32),
  )(sums, cnts, x_stack, wl, wr, b)


_DECR = 1024


def _dec_tc_kernel(gu_ref, gi_ref, w1a_ref, w1b_ref, b1_ref, w2_ref, b2_ref,
                   o_ref):
  t = jnp.dot(gu_ref[...], w1a_ref[...], preferred_element_type=jnp.float32)
  t = t + jnp.dot(gi_ref[...], w1b_ref[...], preferred_element_type=jnp.float32)
  t = jnp.maximum(t + b1_ref[0], 0.0)
  y = jnp.sum(t * w2_ref[0], axis=1, keepdims=True) + b2_ref[0, 0]
  o_ref[...] = y


def _dec_tc(gathered, lin1_W, lin1_b, lin2_W, lin2_b):
  w1a = lin1_W[:D]
  w1b = lin1_W[D:]
  b1 = lin1_b.reshape(1, D)
  w2 = lin2_W.reshape(1, D)
  b2 = lin2_b.reshape(1, 1)
  grid = (pl.cdiv(B, _DECR),)
  off = BH // _DECR
  return pl.pallas_call(
      _dec_tc_kernel,
      grid=grid,
      in_specs=[
          pl.BlockSpec((_DECR, D), lambda i: (i, 0)),
          pl.BlockSpec((_DECR, D), lambda i: (i + off, 0)),
          pl.BlockSpec((D, D), lambda i: (0, 0)),
          pl.BlockSpec((D, D), lambda i: (0, 0)),
          pl.BlockSpec((1, D), lambda i: (0, 0)),
          pl.BlockSpec((1, D), lambda i: (0, 0)),
          pl.BlockSpec(memory_space=pltpu.SMEM),
      ],
      out_specs=pl.BlockSpec((_DECR, 1), lambda i: (i, 0)),
      out_shape=jax.ShapeDtypeStruct((B, 1), jnp.float32),
  )(gathered, gathered, w1a, w1b, b1, w2, b2)


def _pad_to(a, n, val):
  return jnp.concatenate(
      [a, jnp.full((n - a.shape[0],), val, a.dtype)])


def kernel(x_user, x_item, edge_index_ui, edge_index_iu, edge_label_index,
           c1_ui_Wl, c1_ui_Wr, c1_ui_b, c1_iu_Wl, c1_iu_Wr, c1_iu_b,
           c2_ui_Wl, c2_ui_Wr, c2_ui_b, c2_iu_Wl, c2_iu_Wr, c2_iu_b,
           lin1_W, lin1_b, lin2_W, lin2_b):
  i32 = jnp.int32
  iu_s = edge_index_iu[0].astype(i32)
  iu_d = edge_index_iu[1].astype(i32)
  ui_s = edge_index_ui[0].astype(i32)
  ui_d = edge_index_ui[1].astype(i32)

  srcid_flat = jnp.concatenate(
      [_pad_to(iu_s, E_EXT, 0), _pad_to(ui_s, E_EXT, 0)])
  dstid_flat = jnp.concatenate(
      [_pad_to(iu_d, E_EXT, -1), _pad_to(ui_d, E_EXT, -1)])

  x_flat = jnp.concatenate([x_user, x_item], axis=0)
  x_stack = x_flat.reshape(2, N, D)

  wl1 = jnp.stack([c1_iu_Wl, c1_ui_Wl])
  wr1 = jnp.stack([c1_iu_Wr, c1_ui_Wr])
  b1s = jnp.stack([c1_iu_b, c1_ui_b]).reshape(2, 1, D)
  wl2 = jnp.stack([c2_iu_Wl, c2_ui_Wl])
  wr2 = jnp.stack([c2_iu_Wr, c2_ui_Wr])
  b2s = jnp.stack([c2_iu_b, c2_ui_b]).reshape(2, 1, D)

  ones_flat = jnp.ones((GRP, D), jnp.float32)
  cnts = _sage_sc(ones_flat, srcid_flat, dstid_flat, count_only=True)
  sums1 = _sage_sc(x_flat, srcid_flat, dstid_flat)
  h = _sage_tc(sums1, cnts, x_stack, wl1, wr1, b1s, relu=True)

  sums2 = _sage_sc(h.reshape(2 * N, D), srcid_flat, dstid_flat)
  z = _sage_tc(sums2, cnts, h, wl2, wr2, b2s, relu=False)

  idx_u = _pad_to(edge_label_index[0].astype(i32), BH, 0)
  idx_i = _pad_to(edge_label_index[1].astype(i32), BH, 0) + N
  dec_idx = jnp.concatenate([idx_u, idx_i]).reshape(32, DEC_GPW, GRP)

  gathered = _dec_sc(z.reshape(2 * N, D), dec_idx)
  out = _dec_tc(gathered, lin1_W, lin1_b, lin2_W, lin2_b)
  return out.reshape(-1)


# speedup vs baseline: 2.6952x; 1.0003x over previous
import dataclasses
import functools

import jax
import jax.numpy as jnp
from jax import lax
from jax.experimental import pallas as pl
from jax.experimental.pallas import tpu as pltpu
from jax.experimental.pallas import tpu_sc as plsc

N = 50000
D = 128
E = 300000
B = 100000

NTILES = 16
GRP = 128
EPT = 18816
E_PAD = NTILES * EPT
E_EXT = E_PAD + GRP
SEL_CAP = EPT + GRP
NGRP = EPT // GRP
CBLK = 1344
NBLK = EPT // CBLK
CHUNK = 6272
NCHUNK = 8
NP = NCHUNK * CHUNK
DUMMY_SPREAD = 256
ACC_ROWS = CHUNK + DUMMY_SPREAD
ZROWS = ACC_ROWS // NTILES
RPT = CHUNK // NTILES

BH = 106496
DEC_TOT = 2 * BH
DEC_GPW = 52
DEC_NBUF = 4

_mesh = plsc.VectorSubcoreMesh(core_axis_name="c", subcore_axis_name="s")

_sc_params = pltpu.CompilerParams()
if "needs_layout_passes" in pltpu.CompilerParams.__dataclass_fields__:
  _sc_params = dataclasses.replace(_sc_params, needs_layout_passes=False)


def _sage_sc_body(count_only, src_hbm, srcid_hbm, dstid_hbm, zf_hbm,
                  sums_hbm,
                  selv, colb0, colb1,
                  cids0, cids1, sids0, sids1, offs0, offs1, sidx0, sidx1,
                  rows0, rows1,
                  acc_sh,
                  colsem, isem, gsem, fsem, zsem):
  c = lax.axis_index("c")
  s = lax.axis_index("s")
  soff = (1 - c) * N
  ebase = c * E_EXT + s * EPT
  iota = lax.iota(jnp.int32, 16)
  colb = (colb0, colb1)
  cids = (cids0, cids1)
  sids = (sids0, sids1)
  offs = (offs0, offs1)
  sidx = (sidx0, sidx1)
  rows = (rows0, rows1)

  if count_only:
    pltpu.sync_copy(src_hbm.at[pl.ds(0, GRP)], rows[1])

  def col_dma(blk, b):
    src = dstid_hbm.at[pl.ds(ebase + blk * CBLK, CBLK)]
    return pltpu.make_async_copy(src, colb[b], colsem.at[b])

  def idx_dmas(g, b):
    sel = selv.at[pl.ds(g * GRP, GRP)]
    dmas = [pltpu.make_async_copy(dstid_hbm.at[sel], cids[b], isem.at[b])]
    if not count_only:
      dmas.append(
          pltpu.make_async_copy(srcid_hbm.at[sel], sids[b], isem.at[b]))
    return dmas

  def feat_dma(b):
    return pltpu.make_async_copy(src_hbm.at[sidx[b]], rows[b], gsem.at[b])

  def scat_dmas(b):
    srcbuf = rows[1] if count_only else rows[b]
    return (pltpu.make_async_copy(srcbuf, acc_sh.at[offs[b]], fsem.at[b]),)

  @pl.loop(0, NCHUNK)
  def _(chunk):
    base = chunk * CHUNK
    plsc.subcore_barrier()
    zf_cp = pltpu.make_async_copy(zf_hbm, rows[0], zsem)
    zf_cp.start()

    col_dma(jnp.int32(0), 0).start()

    def scan_vec(blk, b):
      def body(i, p):
        col = colb[b][pl.ds(i * 16, 16)]
        u = col - base
        m = (u >= 0) & (u < CHUNK)
        cs = plsc.cumsum(m.astype(jnp.int32))
        ev = (ebase + blk * CBLK) + i * 16 + iota
        plsc.store_scatter(selv, [p + cs - 1], ev, mask=m)
        return p + jnp.sum(m.astype(jnp.int32))
      return body

    def scan_pair(bp, p):
      for b in range(2):
        blk = bp * 2 + b
        col_dma(blk, b).wait()
        nxt = blk + 1

        @pl.when(nxt < NBLK)
        def _():
          col_dma(nxt, 1 - b).start()

        p = lax.fori_loop(0, CBLK // 16, scan_vec(blk, b), p)
      return p

    p = lax.fori_loop(0, NBLK // 2, scan_pair, jnp.int32(0))

    for j in range(GRP // 16):
      selv[pl.ds(p + 16 * j, 16)] = (c * E_EXT + E_PAD + 16 * j) + iota
    ng = (p + (GRP - 1)) // GRP

    zf_cp.wait()
    z0 = s * ZROWS
    for (r0, nr) in ((0, 128), (128, 128), (256, 128), (384, ZROWS - 384)):
      pltpu.async_copy(rows[0].at[pl.ds(0, nr)],
                       acc_sh.at[pl.ds(z0 + r0, nr)], zsem)
    for (r0, nr) in ((0, 128), (128, 128), (256, 128), (384, ZROWS - 384)):
      pltpu.make_async_copy(rows[0].at[pl.ds(0, nr)],
                            acc_sh.at[pl.ds(z0 + r0, nr)], zsem).wait()
    plsc.subcore_barrier()

    for g0 in range(2):
      @pl.when(g0 < ng)
      def _():
        for d in idx_dmas(jnp.int32(g0), g0):
          d.start()

    def step(st, carry):
      for b in range(2):
        g = st * 2 + b

        @pl.when((g >= 2) & (g - 2 < ng))
        def _():
          for d in scat_dmas(b):
            d.wait()

        @pl.when(g < ng)
        def _():
          for d in idx_dmas(g, b):
            d.wait()
          for j in range(GRP // 16):
            cid = cids[b][pl.ds(16 * j, 16)]
            u = cid - base
            ok = (u >= 0) & (u < CHUNK)
            offs[b][pl.ds(16 * j, 16)] = jnp.where(
                ok, u, CHUNK + (cid & (DUMMY_SPREAD - 1)))
            if not count_only:
              sidx[b][pl.ds(16 * j, 16)] = sids[b][pl.ds(16 * j, 16)] + soff
          if not count_only:
            feat_dma(b).start()

        @pl.when(g + 2 < ng)
        def _():
          for d in idx_dmas(g + 2, b):
            d.start()

      for b in range(2):
        g = st * 2 + b

        @pl.when(g < ng)
        def _():
          if not count_only:
            feat_dma(b).wait()
          for d in scat_dmas(b):
            d.start(add=True)
      return carry

    nsteps = (ng + 3) // 2
    lax.fori_loop(0, nsteps, step, jnp.int32(0))

    plsc.subcore_barrier()
    pltpu.sync_copy(acc_sh.at[pl.ds(s * RPT, RPT)],
                    sums_hbm.at[c, pl.ds(base + s * RPT, RPT)])


def _sage_sc(src_flat, srcid_flat, dstid_flat, count_only=False):
  zf = jnp.zeros((GRP, D), jnp.float32)
  kern = pl.kernel(
      functools.partial(_sage_sc_body, count_only),
      out_type=jax.ShapeDtypeStruct((2, NP, D), jnp.float32),
      mesh=_mesh,
      compiler_params=_sc_params,
      scratch_types=[
          pltpu.VMEM((SEL_CAP,), jnp.int32),
          pltpu.VMEM((CBLK,), jnp.int32),
          pltpu.VMEM((CBLK,), jnp.int32),
          pltpu.VMEM((GRP,), jnp.int32),
          pltpu.VMEM((GRP,), jnp.int32),
          pltpu.VMEM((GRP,), jnp.int32),
          pltpu.VMEM((GRP,), jnp.int32),
          pltpu.VMEM((GRP,), jnp.int32),
          pltpu.VMEM((GRP,), jnp.int32),
          pltpu.VMEM((GRP,), jnp.int32),
          pltpu.VMEM((GRP,), jnp.int32),
          pltpu.VMEM((GRP, D), jnp.float32),
          pltpu.VMEM((GRP, D), jnp.float32),
          pltpu.VMEM_SHARED((ACC_ROWS, D), jnp.float32),
          pltpu.SemaphoreType.DMA((2,)),
          pltpu.SemaphoreType.DMA((2,)),
          pltpu.SemaphoreType.DMA((2,)),
          pltpu.SemaphoreType.DMA((2,)),
          pltpu.SemaphoreType.DMA,
      ],
  )
  return kern(src_flat, srcid_flat, dstid_flat, zf)


def _dec_sc_body(z_hbm, idx_hbm, out_hbm, idx_v, rows0, rows1, rows2, rows3,
                 gsem, osem):
  c = lax.axis_index("c")
  s = lax.axis_index("s")
  w = s * 2 + c
  rows = (rows0, rows1, rows2, rows3)
  pltpu.sync_copy(idx_hbm.at[w], idx_v)

  def gather_dma(g, b):
    return pltpu.make_async_copy(z_hbm.at[idx_v.at[g]], rows[b], gsem.at[b])

  def out_dma(g, b):
    dst = out_hbm.at[pl.ds(w * (DEC_GPW * GRP) + g * GRP, GRP)]
    return pltpu.make_async_copy(rows[b], dst, osem.at[b])

  for b in range(DEC_NBUF):
    gather_dma(jnp.int32(b), b).start()

  @pl.loop(0, DEC_GPW // DEC_NBUF)
  def _(it):
    for b in range(DEC_NBUF):
      g = it * DEC_NBUF + b
      gather_dma(g, b).wait()
      out_dma(g, b).start()
    for b in range(DEC_NBUF):
      g = it * DEC_NBUF + b
      out_dma(g, b).wait()
      g_next = g + DEC_NBUF

      @pl.when(g_next < DEC_GPW)
      def _():
        gather_dma(g_next, b).start()


def _dec_sc(z_flat, dec_idx):
  kern = pl.kernel(
      _dec_sc_body,
      out_type=jax.ShapeDtypeStruct((DEC_TOT, D), jnp.float32),
      mesh=_mesh,
      scratch_types=[
          pltpu.VMEM((DEC_GPW, GRP), jnp.int32),
          pltpu.VMEM((GRP, D), jnp.float32),
          pltpu.VMEM((GRP, D), jnp.float32),
          pltpu.VMEM((GRP, D), jnp.float32),
          pltpu.VMEM((GRP, D), jnp.float32),
          pltpu.SemaphoreType.DMA((DEC_NBUF,)),
          pltpu.SemaphoreType.DMA((DEC_NBUF,)),
      ],
  )
  return kern(z_flat, dec_idx)


_TCR = 1000


def _sage_tc_kernel(relu, sum_ref, cnt_ref, x_ref, wl_ref, wr_ref, b_ref,
                    o_ref):
  cnt = cnt_ref[0, :, 0:1]
  inv = 1.0 / jnp.maximum(cnt, 1.0)
  mean = sum_ref[0] * inv
  acc = jnp.dot(mean, wl_ref[0], preferred_element_type=jnp.float32)
  acc = acc + jnp.dot(x_ref[0], wr_ref[0], preferred_element_type=jnp.float32)
  acc = acc + b_ref[0, 0]
  if relu:
    acc = jnp.maximum(acc, 0.0)
  o_ref[0] = acc


def _sage_tc(sums, cnts, x_stack, wl, wr, b, relu):
  grid = (2, N // _TCR)
  return pl.pallas_call(
      functools.partial(_sage_tc_kernel, relu),
      grid=grid,
      in_specs=[
          pl.BlockSpec((1, _TCR, D), lambda t, i: (t, i, 0)),
          pl.BlockSpec((1, _TCR, D), lambda t, i: (t, i, 0)),
          pl.BlockSpec((1, _TCR, D), lambda t, i: (t, i, 0)),
          pl.BlockSpec((1, D, D), lambda t, i: (t, 0, 0)),
          pl.BlockSpec((1, D, D), lambda t, i: (t, 0, 0)),
          pl.BlockSpec((1, 1, D), lambda t, i: (t, 0, 0)),
      ],
      out_specs=pl.BlockSpec((1, _TCR, D), lambda t, i: (t, i, 0)),
      out_shape=jax.ShapeDtypeStruct((2, N, D), jnp.float32),
  )(sums, cnts, x_stack, wl, wr, b)


_DECR = 1024


def _dec_tc_kernel(gu_ref, gi_ref, w1a_ref, w1b_ref, b1_ref, w2_ref, b2_ref,
                   o_ref):
  t = jnp.dot(gu_ref[...], w1a_ref[...], preferred_element_type=jnp.float32)
  t = t + jnp.dot(gi_ref[...], w1b_ref[...], preferred_element_type=jnp.float32)
  t = jnp.maximum(t + b1_ref[0], 0.0)
  y = jnp.sum(t * w2_ref[0], axis=1, keepdims=True) + b2_ref[0, 0]
  o_ref[...] = y


def _dec_tc(gathered, lin1_W, lin1_b, lin2_W, lin2_b):
  w1a = lin1_W[:D]
  w1b = lin1_W[D:]
  b1 = lin1_b.reshape(1, D)
  w2 = lin2_W.reshape(1, D)
  b2 = lin2_b.reshape(1, 1)
  grid = (pl.cdiv(B, _DECR),)
  off = BH // _DECR
  return pl.pallas_call(
      _dec_tc_kernel,
      grid=grid,
      in_specs=[
          pl.BlockSpec((_DECR, D), lambda i: (i, 0)),
          pl.BlockSpec((_DECR, D), lambda i: (i + off, 0)),
          pl.BlockSpec((D, D), lambda i: (0, 0)),
          pl.BlockSpec((D, D), lambda i: (0, 0)),
          pl.BlockSpec((1, D), lambda i: (0, 0)),
          pl.BlockSpec((1, D), lambda i: (0, 0)),
          pl.BlockSpec(memory_space=pltpu.SMEM),
      ],
      out_specs=pl.BlockSpec((_DECR, 1), lambda i: (i, 0)),
      out_shape=jax.ShapeDtypeStruct((B, 1), jnp.float32),
  )(gathered, gathered, w1a, w1b, b1, w2, b2)


def _pad_to(a, n, val):
  return jnp.concatenate(
      [a, jnp.full((n - a.shape[0],), val, a.dtype)])


def kernel(x_user, x_item, edge_index_ui, edge_index_iu, edge_label_index,
           c1_ui_Wl, c1_ui_Wr, c1_ui_b, c1_iu_Wl, c1_iu_Wr, c1_iu_b,
           c2_ui_Wl, c2_ui_Wr, c2_ui_b, c2_iu_Wl, c2_iu_Wr, c2_iu_b,
           lin1_W, lin1_b, lin2_W, lin2_b):
  i32 = jnp.int32
  iu_s = edge_index_iu[0].astype(i32)
  iu_d = edge_index_iu[1].astype(i32)
  ui_s = edge_index_ui[0].astype(i32)
  ui_d = edge_index_ui[1].astype(i32)

  srcid_flat = jnp.concatenate(
      [_pad_to(iu_s, E_EXT, 0), _pad_to(ui_s, E_EXT, 0)])
  dstid_flat = jnp.concatenate(
      [_pad_to(iu_d, E_EXT, -1), _pad_to(ui_d, E_EXT, -1)])

  x_flat = jnp.concatenate([x_user, x_item], axis=0)
  x_stack = x_flat.reshape(2, N, D)

  wl1 = jnp.stack([c1_iu_Wl, c1_ui_Wl])
  wr1 = jnp.stack([c1_iu_Wr, c1_ui_Wr])
  b1s = jnp.stack([c1_iu_b, c1_ui_b]).reshape(2, 1, D)
  wl2 = jnp.stack([c2_iu_Wl, c2_ui_Wl])
  wr2 = jnp.stack([c2_iu_Wr, c2_ui_Wr])
  b2s = jnp.stack([c2_iu_b, c2_ui_b]).reshape(2, 1, D)

  ones_flat = jnp.ones((GRP, D), jnp.float32)
  cnts = _sage_sc(ones_flat, srcid_flat, dstid_flat, count_only=True)
  sums1 = _sage_sc(x_flat, srcid_flat, dstid_flat)
  h = _sage_tc(sums1, cnts, x_stack, wl1, wr1, b1s, relu=True)

  sums2 = _sage_sc(h.reshape(2 * N, D), srcid_flat, dstid_flat)
  z = _sage_tc(sums2, cnts, h, wl2, wr2, b2s, relu=False)

  idx_u = _pad_to(edge_label_index[0].astype(i32), BH, 0)
  idx_i = _pad_to(edge_label_index[1].astype(i32), BH, 0) + N
  dec_idx = jnp.concatenate([idx_u, idx_i]).reshape(32, DEC_GPW, GRP)

  gathered = _dec_sc(z.reshape(2 * N, D), dec_idx)
  out = _dec_tc(gathered, lin1_W, lin1_b, lin2_W, lin2_b)
  return out.reshape(-1)
